# Initial kernel scaffold; baseline (speedup 1.0000x reference)
import jax
import jax.numpy as jnp
from jax.experimental import pallas as pl


def kernel(adj_indices, adj_values, features, weight_matrix, bias, ln_gamma, ln_beta):
    raise NotImplementedError("write your pallas kernel here")


# speedup vs baseline: 3.2864x; 3.2864x over previous
import functools

import jax
import jax.numpy as jnp
from jax import lax
from jax.experimental import pallas as pl
from jax.experimental.pallas import tpu as pltpu
from jax.experimental.pallas import tpu_sc as plsc

NC = 2
NS = 16
CH = 128


def _matmul(features, weight_matrix):
    n, d_in = features.shape
    d_out = weight_matrix.shape[1]
    bm = 2000
    grid = (n // bm,)

    def body(x_ref, w_ref, o_ref):
        o_ref[...] = jnp.dot(x_ref[...], w_ref[...],
                             preferred_element_type=jnp.float32)

    return pl.pallas_call(
        body,
        grid=grid,
        in_specs=[
            pl.BlockSpec((bm, d_in), lambda i: (i, 0)),
            pl.BlockSpec((d_in, d_out), lambda i: (0, 0)),
        ],
        out_specs=pl.BlockSpec((bm, d_out), lambda i: (i, 0)),
        out_shape=jax.ShapeDtypeStruct((n, d_out), jnp.float32),
    )(features, weight_matrix)


def _add_partials(p, n):
    d = p.shape[2]
    bm = 2000

    def body(p_ref, o_ref):
        o_ref[...] = p_ref[0] + p_ref[1]

    return pl.pallas_call(
        body,
        grid=(n // bm,),
        in_specs=[pl.BlockSpec((2, bm, d), lambda i: (0, i, 0))],
        out_specs=pl.BlockSpec((bm, d), lambda i: (i, 0)),
        out_shape=jax.ShapeDtypeStruct((n, d), jnp.float32),
    )(p)


def _finalize(p, bias, ln_gamma, ln_beta, n):
    d = p.shape[2]
    bm = 2000

    def body(p_ref, b_ref, g_ref, t_ref, o_ref):
        x = p_ref[0] + p_ref[1] + b_ref[...]
        mean = jnp.mean(x, axis=-1, keepdims=True)
        cent = x - mean
        var = jnp.mean(cent * cent, axis=-1, keepdims=True)
        o_ref[...] = cent * lax.rsqrt(var + 1e-5) * g_ref[...] + t_ref[...]

    return pl.pallas_call(
        body,
        grid=(n // bm,),
        in_specs=[
            pl.BlockSpec((2, bm, d), lambda i: (0, i, 0)),
            pl.BlockSpec((1, d), lambda i: (0, 0)),
            pl.BlockSpec((1, d), lambda i: (0, 0)),
            pl.BlockSpec((1, d), lambda i: (0, 0)),
        ],
        out_specs=pl.BlockSpec((bm, d), lambda i: (i, 0)),
        out_shape=jax.ShapeDtypeStruct((n, d), jnp.float32),
    )(p, bias, ln_gamma, ln_beta)


@functools.lru_cache(maxsize=None)
def _make_spmm(n, d, ep):
    cpw = ep // (NC * NS * CH)
    rpt = (((n + NS - 1) // NS) + 7) // 8 * 8
    np_ = rpt * NS
    epw = cpw * CH
    lanes_per_row = d // 16

    def body(base_hbm, row_hbm, col_hbm, val_hbm, out_hbm,
             rowv, colv, valv, rows, acc, sem):
        cid = lax.axis_index("c")
        sid = lax.axis_index("s")
        w = cid * NS + sid

        zero16 = jnp.zeros((16,), jnp.float32)

        def zbody(e, _):
            for j in range(lanes_per_row):
                rows[e, pl.ds(j * 16, 16)] = zero16
            return 0

        lax.fori_loop(0, CH, zbody, 0)

        r0 = sid * rpt
        full = rpt // CH
        rem = rpt - full * CH
        for t in range(full):
            pltpu.sync_copy(rows, acc.at[pl.ds(r0 + t * CH, CH)])
        if rem:
            pltpu.sync_copy(rows.at[pl.ds(0, rem)],
                            acc.at[pl.ds(r0 + full * CH, rem)])
        plsc.subcore_barrier()

        def chunk_body(k, _):
            off = w * epw + k * CH
            pltpu.sync_copy(row_hbm.at[pl.ds(off, CH)], rowv)
            pltpu.sync_copy(col_hbm.at[pl.ds(off, CH)], colv)
            pltpu.sync_copy(val_hbm.at[pl.ds(off, CH)], valv)
            pltpu.async_copy(base_hbm.at[colv], rows, sem).wait()

            def scale(e, _2):
                v = plsc.load_gather(valv, [lax.broadcast(e, (16,))])
                for j in range(lanes_per_row):
                    sl = pl.ds(j * 16, 16)
                    rows[e, sl] = rows[e, sl] * v
                return 0

            lax.fori_loop(0, CH, scale, 0)
            pltpu.sync_copy(rows, acc.at[rowv], add=True)
            return 0

        lax.fori_loop(0, cpw, chunk_body, 0)
        plsc.subcore_barrier()
        pltpu.sync_copy(acc.at[pl.ds(r0, rpt)],
                        out_hbm.at[cid, pl.ds(r0, rpt)])

    mesh = plsc.VectorSubcoreMesh(core_axis_name="c", subcore_axis_name="s")
    return pl.kernel(
        body,
        out_type=jax.ShapeDtypeStruct((NC, np_, d), jnp.float32),
        mesh=mesh,
        scratch_types=[
            pltpu.VMEM((CH,), jnp.int32),
            pltpu.VMEM((CH,), jnp.int32),
            pltpu.VMEM((CH,), jnp.float32),
            pltpu.VMEM((CH, d), jnp.float32),
            pltpu.VMEM_SHARED((np_, d), jnp.float32),
            pltpu.SemaphoreType.DMA,
        ],
        compiler_params=pltpu.CompilerParams(needs_layout_passes=False),
    )


def kernel(adj_indices, adj_values, features, weight_matrix, bias,
           ln_gamma, ln_beta):
    n, d_in = features.shape
    d = weight_matrix.shape[1]
    e = adj_values.shape[0]

    row = adj_indices[0].astype(jnp.int32)
    col = adj_indices[1].astype(jnp.int32)
    val = adj_values.astype(jnp.float32)

    per = NC * NS * CH
    ep = ((e + per - 1) // per) * per
    if ep != e:
        pad = ep - e
        row = jnp.concatenate([row, jnp.zeros((pad,), jnp.int32)])
        col = jnp.concatenate([col, jnp.zeros((pad,), jnp.int32)])
        val = jnp.concatenate([val, jnp.zeros((pad,), jnp.float32)])

    spmm = _make_spmm(n, d, ep)

    base = _matmul(features, weight_matrix)
    p = spmm(base, row, col, val)
    base = _add_partials(p, n)
    p = spmm(base, row, col, val)

    bias2 = jnp.reshape(bias, (1, d))
    gamma2 = jnp.reshape(ln_gamma, (1, d))
    beta2 = jnp.reshape(ln_beta, (1, d))
    return _finalize(p, bias2, gamma2, beta2, n)
